# baseline (speedup 1.0000x reference)
import functools

import jax
import jax.numpy as jnp
from jax import lax
from jax.experimental import pallas as pl
from jax.experimental.pallas import tpu as pltpu
from jax.experimental.pallas import tpu_sc as plsc

N_NODES = 10000
D_FEAT = 128
D_HID = 512
D_MLP = 256
N_CLASS = 5
N_EDGES = 320000
NC = 2
NS = 16
KP = D_HID // 128
EDGE_CHUNK = 64
WCHUNKS = 40
NWIN = 4
CHUNKS = WCHUNKS * NWIN
E_TILE = CHUNKS * EDGE_CHUNK
E_PAD = NC * NS * E_TILE
N_PAD = 10112
ROWS_PER_TILE = N_PAD // NS
ZCOPIES = ROWS_PER_TILE // EDGE_CHUNK
ZTAIL = ROWS_PER_TILE - ZCOPIES * EDGE_CHUNK


def _sc_scatter_body(h0v_hbm, esrc4_hbm, edst_hbm, out_hbm,
                     src0, src1, dst0, dst1, rows0, rows1, acc,
                     gsem0, gsem1, wsem):
    c = lax.axis_index("c")
    s = lax.axis_index("s")
    wid = s * NC + c
    srcw = (src0, src1)
    dstw = (dst0, dst1)

    def stage_window(k, w, sync):
        sb, db = srcw[w % 2], dstw[w % 2]
        if sync:
            pltpu.sync_copy(esrc4_hbm.at[k, wid, pl.ds(w * WCHUNKS, WCHUNKS)],
                            sb)
            pltpu.sync_copy(edst_hbm.at[wid, pl.ds(w * WCHUNKS, WCHUNKS)], db)
        else:
            pltpu.async_copy(esrc4_hbm.at[k, wid, pl.ds(w * WCHUNKS, WCHUNKS)],
                             sb, wsem)
            pltpu.async_copy(edst_hbm.at[wid, pl.ds(w * WCHUNKS, WCHUNKS)],
                             db, wsem)

    def wait_window(k, w):
        sb, db = srcw[w % 2], dstw[w % 2]
        pltpu.make_async_copy(esrc4_hbm.at[k, wid, pl.ds(0, WCHUNKS)],
                              sb, wsem).wait()
        pltpu.make_async_copy(edst_hbm.at[wid, pl.ds(0, WCHUNKS)],
                              db, wsem).wait()

    for k in range(KP):
        stage_window(k, 0, sync=True)
        stage_window(k, 1, sync=False)

        def _zb(i, carry):
            r = i // 8
            col = (i % 8) * 16
            rows0[r, pl.ds(col, 16)] = jnp.zeros((16,), jnp.float32)
            return carry

        lax.fori_loop(0, EDGE_CHUNK * 8, _zb, 0)

        def _zacc(i, carry):
            pltpu.sync_copy(
                rows0,
                acc.at[pl.ds(s * ROWS_PER_TILE + i * EDGE_CHUNK, EDGE_CHUNK)])
            return carry

        lax.fori_loop(0, ZCOPIES, _zacc, 0)
        pltpu.sync_copy(
            rows0.at[pl.ds(0, ZTAIL)],
            acc.at[pl.ds(s * ROWS_PER_TILE + ZCOPIES * EDGE_CHUNK, ZTAIL)])

        plsc.subcore_barrier()

        for w in range(NWIN):
            sb, db = srcw[w % 2], dstw[w % 2]
            if w > 0:
                wait_window(k, w)
            if w < NWIN - 1:
                stage_window(k, w + 1, sync=False)

            pltpu.async_copy(h0v_hbm.at[sb.at[0]], rows0, gsem0)

            def _pair(j2, carry):
                j = 2 * j2
                pltpu.make_async_copy(h0v_hbm.at[sb.at[0]], rows0,
                                      gsem0).wait()
                pltpu.async_copy(h0v_hbm.at[sb.at[j + 1]], rows1, gsem1)
                pltpu.sync_copy(rows0, acc.at[db.at[j]], add=True)
                pltpu.make_async_copy(h0v_hbm.at[sb.at[0]], rows1,
                                      gsem1).wait()
                pltpu.async_copy(h0v_hbm.at[sb.at[j + 2]], rows0, gsem0)
                pltpu.sync_copy(rows1, acc.at[db.at[j + 1]], add=True)
                return carry

            lax.fori_loop(0, WCHUNKS // 2 - 1, _pair, 0)

            j = WCHUNKS - 2
            pltpu.make_async_copy(h0v_hbm.at[sb.at[0]], rows0, gsem0).wait()
            pltpu.async_copy(h0v_hbm.at[sb.at[j + 1]], rows1, gsem1)
            pltpu.sync_copy(rows0, acc.at[db.at[j]], add=True)
            pltpu.make_async_copy(h0v_hbm.at[sb.at[0]], rows1, gsem1).wait()
            pltpu.sync_copy(rows1, acc.at[db.at[j + 1]], add=True)

        plsc.subcore_barrier()

        r0 = s * ROWS_PER_TILE
        pltpu.sync_copy(acc.at[pl.ds(r0, ROWS_PER_TILE)],
                        out_hbm.at[k, c, pl.ds(r0, ROWS_PER_TILE)])


_sc_scatter = functools.partial(
    pl.kernel,
    out_type=jax.ShapeDtypeStruct((KP, NC, N_PAD, 128), jnp.float32),
    mesh=plsc.VectorSubcoreMesh(core_axis_name="c", subcore_axis_name="s"),
    scratch_types=[
        pltpu.VMEM((WCHUNKS, EDGE_CHUNK), jnp.int32),
        pltpu.VMEM((WCHUNKS, EDGE_CHUNK), jnp.int32),
        pltpu.VMEM((WCHUNKS, EDGE_CHUNK), jnp.int32),
        pltpu.VMEM((WCHUNKS, EDGE_CHUNK), jnp.int32),
        pltpu.VMEM((EDGE_CHUNK, 128), jnp.float32),
        pltpu.VMEM((EDGE_CHUNK, 128), jnp.float32),
        pltpu.VMEM_SHARED((N_PAD, 128), jnp.float32),
        pltpu.SemaphoreType.DMA,
        pltpu.SemaphoreType.DMA,
        pltpu.SemaphoreType.DMA,
    ],
)(_sc_scatter_body)


def _h0_body(nodes_ref, w_in_ref, b_in_ref, out_ref):
    out_ref[...] = jnp.dot(nodes_ref[...], w_in_ref[...],
                           preferred_element_type=jnp.float32) + b_in_ref[...]


_tc_h0 = pl.pallas_call(
    _h0_body,
    out_shape=jax.ShapeDtypeStruct((N_NODES, D_HID), jnp.float32),
)


ROW_BLK = 2000
N_BLKS = N_NODES // ROW_BLK


def _tc_body(h0_ref, part_ref, w_gcn_ref, b_gcn_ref,
             w1_ref, b1_ref, w2_ref, b2_ref, out_ref):
    i = pl.program_id(0)

    h0 = h0_ref[...]
    agg = jnp.concatenate(
        [part_ref[k, 0] + part_ref[k, 1] for k in range(KP)], axis=1)
    feature = jnp.maximum(
        jnp.dot(agg, w_gcn_ref[...], preferred_element_type=jnp.float32)
        + b_gcn_ref[...], 0.0) + h0
    x = jnp.maximum(
        jnp.dot(feature, w1_ref[...], preferred_element_type=jnp.float32)
        + b1_ref[...], 0.0)
    logits = jnp.dot(x, w2_ref[...],
                     preferred_element_type=jnp.float32) + b2_ref[...]
    out_ref[pl.ds(i * ROW_BLK, ROW_BLK), :] = logits

    @pl.when(i == N_BLKS - 1)
    def _():
        lg = out_ref[...]
        m = jnp.max(lg, axis=0, keepdims=True)
        e = jnp.exp(lg - m)
        out_ref[...] = e / jnp.sum(e, axis=0, keepdims=True)


def _full(shape):
    return pl.BlockSpec(shape, lambda i: (0,) * len(shape))


_tc_dense = pl.pallas_call(
    _tc_body,
    grid=(N_BLKS,),
    in_specs=[
        pl.BlockSpec((ROW_BLK, D_HID), lambda i: (i, 0)),
        pl.BlockSpec((KP, NC, ROW_BLK, 128), lambda i: (0, 0, i, 0)),
        _full((D_HID, D_HID)),
        _full((1, D_HID)),
        _full((D_HID, D_MLP)),
        _full((1, D_MLP)),
        _full((D_MLP, N_CLASS)),
        _full((1, N_CLASS)),
    ],
    out_specs=_full((N_NODES, N_CLASS)),
    out_shape=jax.ShapeDtypeStruct((N_NODES, N_CLASS), jnp.float32),
)


def kernel(nodes, edges, W_in, b_in, W_gcn, b_gcn, W1, b1, W2, b2):
    src = edges[0].astype(jnp.int32)
    dst = edges[1].astype(jnp.int32)
    pad = E_PAD - N_EDGES
    src_p = jnp.concatenate([src * 4, jnp.zeros((pad,), jnp.int32)])
    dst_p = jnp.concatenate([dst, jnp.full((pad,), N_NODES, jnp.int32)])
    esrc4 = (src_p[None, :] + jnp.arange(KP, dtype=jnp.int32)[:, None]
             ).reshape(KP, NC * NS, CHUNKS, EDGE_CHUNK)
    edst = dst_p.reshape(NC * NS, CHUNKS, EDGE_CHUNK)

    h0 = _tc_h0(nodes, W_in, b_in.reshape(1, D_HID))
    h0v = h0.reshape(KP * N_NODES, 128)
    partials = _sc_scatter(h0v, esrc4, edst)
    return _tc_dense(h0, partials,
                     W_gcn, b_gcn.reshape(1, D_HID),
                     W1, b1.reshape(1, 256),
                     W2, b2.reshape(1, N_CLASS))


# speedup vs baseline: 2.9685x; 2.9685x over previous
import functools

import jax
import jax.numpy as jnp
from jax import lax
from jax.experimental import pallas as pl
from jax.experimental.pallas import tpu as pltpu
from jax.experimental.pallas import tpu_sc as plsc

N_NODES = 10000
D_FEAT = 128
D_HID = 512
D_MLP = 256
N_CLASS = 5
N_EDGES = 320000
NC = 2
NS = 16
KP = D_HID // 128
EDGE_CHUNK = 80
WCHUNKS = 25
NWIN = 5
CHUNKS = WCHUNKS * NWIN
E_TILE = CHUNKS * EDGE_CHUNK
E_PAD = NC * NS * E_TILE
N_PAD = 10112
ROWS_PER_TILE = N_PAD // NS
ZCOPIES = ROWS_PER_TILE // EDGE_CHUNK
ZTAIL = ROWS_PER_TILE - ZCOPIES * EDGE_CHUNK


def _sc_scatter_body(h0v_hbm, esrc4_hbm, edst_hbm, out_hbm,
                     src0, src1, dst0, dst1, rows0, rows1, acc,
                     gsem0, gsem1, wsem):
    c = lax.axis_index("c")
    s = lax.axis_index("s")
    wid = s * NC + c
    srcw = (src0, src1)
    dstw = (dst0, dst1)

    def stage_window(k, w, sync):
        sb, db = srcw[w % 2], dstw[w % 2]
        if sync:
            pltpu.sync_copy(esrc4_hbm.at[k, wid, w], sb)
            pltpu.sync_copy(edst_hbm.at[wid, w], db)
        else:
            pltpu.async_copy(esrc4_hbm.at[k, wid, w], sb, wsem)
            pltpu.async_copy(edst_hbm.at[wid, w], db, wsem)

    def wait_window(k, w):
        sb, db = srcw[w % 2], dstw[w % 2]
        pltpu.make_async_copy(esrc4_hbm.at[k, wid, 0], sb, wsem).wait()
        pltpu.make_async_copy(edst_hbm.at[wid, 0], db, wsem).wait()

    for k in range(KP):
        stage_window(k, 0, sync=True)
        stage_window(k, 1, sync=False)

        def _zb(i, carry):
            r = i // 8
            col = (i % 8) * 16
            rows0[r, pl.ds(col, 16)] = jnp.zeros((16,), jnp.float32)
            return carry

        lax.fori_loop(0, EDGE_CHUNK * 8, _zb, 0)

        def _zacc(i, carry):
            pltpu.sync_copy(
                rows0,
                acc.at[pl.ds(s * ROWS_PER_TILE + i * EDGE_CHUNK, EDGE_CHUNK)])
            return carry

        lax.fori_loop(0, ZCOPIES, _zacc, 0)
        pltpu.sync_copy(
            rows0.at[pl.ds(0, ZTAIL)],
            acc.at[pl.ds(s * ROWS_PER_TILE + ZCOPIES * EDGE_CHUNK, ZTAIL)])

        plsc.subcore_barrier()

        for w in range(NWIN):
            sb, db = srcw[w % 2], dstw[w % 2]
            if w > 0:
                wait_window(k, w)
            if w < NWIN - 1:
                stage_window(k, w + 1, sync=False)

            pltpu.sync_copy(h0v_hbm.at[sb.at[0]], rows0)

            def _pair(j2, carry):
                j = 2 * j2
                d = pltpu.async_copy(h0v_hbm.at[sb.at[j + 1]], rows1, gsem0)
                pltpu.sync_copy(rows0, acc.at[db.at[j]], add=True)
                d.wait()
                d = pltpu.async_copy(h0v_hbm.at[sb.at[j + 2]], rows0, gsem1)
                pltpu.sync_copy(rows1, acc.at[db.at[j + 1]], add=True)
                d.wait()
                return carry

            lax.fori_loop(0, WCHUNKS // 2, _pair, 0)

            pltpu.sync_copy(rows0, acc.at[db.at[WCHUNKS - 1]], add=True)

        plsc.subcore_barrier()

        r0 = s * ROWS_PER_TILE
        pltpu.sync_copy(acc.at[pl.ds(r0, ROWS_PER_TILE)],
                        out_hbm.at[k, c, pl.ds(r0, ROWS_PER_TILE)])


_sc_scatter = functools.partial(
    pl.kernel,
    out_type=jax.ShapeDtypeStruct((KP, NC, N_PAD, 128), jnp.float32),
    mesh=plsc.VectorSubcoreMesh(core_axis_name="c", subcore_axis_name="s"),
    scratch_types=[
        pltpu.VMEM((WCHUNKS, EDGE_CHUNK), jnp.int32),
        pltpu.VMEM((WCHUNKS, EDGE_CHUNK), jnp.int32),
        pltpu.VMEM((WCHUNKS, EDGE_CHUNK), jnp.int32),
        pltpu.VMEM((WCHUNKS, EDGE_CHUNK), jnp.int32),
        pltpu.VMEM((EDGE_CHUNK, 128), jnp.float32),
        pltpu.VMEM((EDGE_CHUNK, 128), jnp.float32),
        pltpu.VMEM_SHARED((N_PAD, 128), jnp.float32),
        pltpu.SemaphoreType.DMA,
        pltpu.SemaphoreType.DMA,
        pltpu.SemaphoreType.DMA,
    ],
)(_sc_scatter_body)


def _h0_body(nodes_ref, w_in_ref, b_in_ref, out_ref):
    out_ref[...] = jnp.dot(nodes_ref[...], w_in_ref[...],
                           preferred_element_type=jnp.float32) + b_in_ref[...]


_tc_h0 = pl.pallas_call(
    _h0_body,
    out_shape=jax.ShapeDtypeStruct((N_NODES, D_HID), jnp.float32),
)


ROW_BLK = 2000
N_BLKS = N_NODES // ROW_BLK


def _tc_body(h0_ref, part_ref, w_gcn_ref, b_gcn_ref,
             w1_ref, b1_ref, w2_ref, b2_ref, out_ref):
    i = pl.program_id(0)

    h0 = h0_ref[...]
    agg = jnp.concatenate(
        [part_ref[k, 0] + part_ref[k, 1] for k in range(KP)], axis=1)
    feature = jnp.maximum(
        jnp.dot(agg, w_gcn_ref[...], preferred_element_type=jnp.float32)
        + b_gcn_ref[...], 0.0) + h0
    x = jnp.maximum(
        jnp.dot(feature, w1_ref[...], preferred_element_type=jnp.float32)
        + b1_ref[...], 0.0)
    logits = jnp.dot(x, w2_ref[...],
                     preferred_element_type=jnp.float32) + b2_ref[...]
    out_ref[pl.ds(i * ROW_BLK, ROW_BLK), :] = logits

    @pl.when(i == N_BLKS - 1)
    def _():
        lg = out_ref[...]
        m = jnp.max(lg, axis=0, keepdims=True)
        e = jnp.exp(lg - m)
        out_ref[...] = e / jnp.sum(e, axis=0, keepdims=True)


def _full(shape):
    return pl.BlockSpec(shape, lambda i: (0,) * len(shape))


_tc_dense = pl.pallas_call(
    _tc_body,
    grid=(N_BLKS,),
    in_specs=[
        pl.BlockSpec((ROW_BLK, D_HID), lambda i: (i, 0)),
        pl.BlockSpec((KP, NC, ROW_BLK, 128), lambda i: (0, 0, i, 0)),
        _full((D_HID, D_HID)),
        _full((1, D_HID)),
        _full((D_HID, D_MLP)),
        _full((1, D_MLP)),
        _full((D_MLP, N_CLASS)),
        _full((1, N_CLASS)),
    ],
    out_specs=_full((N_NODES, N_CLASS)),
    out_shape=jax.ShapeDtypeStruct((N_NODES, N_CLASS), jnp.float32),
)


def kernel(nodes, edges, W_in, b_in, W_gcn, b_gcn, W1, b1, W2, b2):
    src = edges[0].astype(jnp.int32)
    dst = edges[1].astype(jnp.int32)
    pad = E_PAD - N_EDGES
    src_p = jnp.concatenate([src * 4, jnp.zeros((pad,), jnp.int32)])
    dst_p = jnp.concatenate([dst, jnp.full((pad,), N_NODES, jnp.int32)])
    esrc4 = (src_p[None, :] + jnp.arange(KP, dtype=jnp.int32)[:, None]
             ).reshape(KP, NC * NS, NWIN, WCHUNKS, EDGE_CHUNK)
    edst = dst_p.reshape(NC * NS, NWIN, WCHUNKS, EDGE_CHUNK)

    h0 = _tc_h0(nodes, W_in, b_in.reshape(1, D_HID))
    h0v = h0.reshape(KP * N_NODES, 128)
    partials = _sc_scatter(h0v, esrc4, edst)
    return _tc_dense(h0, partials,
                     W_gcn, b_gcn.reshape(1, D_HID),
                     W1, b1.reshape(1, 256),
                     W2, b2.reshape(1, N_CLASS))
